# Initial kernel scaffold; baseline (speedup 1.0000x reference)
import jax
import jax.numpy as jnp
from jax.experimental import pallas as pl


def kernel(node_features, node_to_graph_id, W_proj, b_proj, W_gate, b_gate):
    raise NotImplementedError("write your pallas kernel here")


# speedup vs baseline: 8.6097x; 8.6097x over previous
import jax
import jax.numpy as jnp
from jax.experimental import pallas as pl

N, D, H, G = 50000, 512, 512, 512
R = 2000
GRID = N // R


def _fused_body(ids_ref, x_ref, wp_ref, bp_ref, wg_ref, bg_ref, out_ref):
    i = pl.program_id(0)
    x = x_ref[...]
    proj = jnp.dot(x, wp_ref[...], preferred_element_type=jnp.float32) + bp_ref[...]
    gate_l = jnp.dot(x, wg_ref[...], preferred_element_type=jnp.float32) + bg_ref[...]
    gated = jax.nn.sigmoid(gate_l) * proj
    ids = ids_ref[0]
    onehot_t = (jax.lax.broadcasted_iota(jnp.int32, (G, R), 0) == ids
                ).astype(jnp.float32)
    partial = jnp.dot(onehot_t, gated, preferred_element_type=jnp.float32)

    @pl.when(i == 0)
    def _init():
        out_ref[...] = jnp.zeros_like(out_ref)

    out_ref[...] += partial


def kernel(node_features, node_to_graph_id, W_proj, b_proj, W_gate, b_gate):
    ids3 = node_to_graph_id.astype(jnp.int32).reshape(GRID, 1, R)
    bp2 = b_proj.reshape(1, H)
    bg2 = b_gate.reshape(1, H)
    return pl.pallas_call(
        _fused_body,
        grid=(GRID,),
        in_specs=[
            pl.BlockSpec((1, 1, R), lambda i: (i, 0, 0)),
            pl.BlockSpec((R, D), lambda i: (i, 0)),
            pl.BlockSpec((D, H), lambda i: (0, 0)),
            pl.BlockSpec((1, H), lambda i: (0, 0)),
            pl.BlockSpec((D, H), lambda i: (0, 0)),
            pl.BlockSpec((1, H), lambda i: (0, 0)),
        ],
        out_specs=pl.BlockSpec((G, H), lambda i: (0, 0)),
        out_shape=jax.ShapeDtypeStruct((G, H), jnp.float32),
    )(ids3, node_features, W_proj, bp2, W_gate, bg2)
